# baseline (speedup 1.0000x reference)
import functools

import jax
import jax.numpy as jnp
from jax import lax
from jax.experimental import pallas as pl
from jax.experimental.pallas import tpu as pltpu
from jax.experimental.pallas import tpu_sc as plsc

NN = 10000
NE = 320000
HID = 128
NC, NS, LANES = 2, 16, 16
NW = NC * NS
EB = 512
NBLK = NE // EB
GPB = EB // LANES
IDXROWS = EB // 128
NPAD = 16384
DPL = 3 * NPAD
ZCH = DPL // NS
POSPAD = 30720
NB = NPAD // NW

_MESH = plsc.VectorSubcoreMesh(
    core_axis_name="c", subcore_axis_name="s", num_cores=NC, num_subcores=NS)
_PARAMS = pltpu.CompilerParams(needs_layout_passes=False)


def _rsqrt(x):
    xi = lax.bitcast_convert_type(x, jnp.int32)
    yi = jnp.int32(0x5F3759DF) - lax.shift_right_logical(xi, 1)
    y = lax.bitcast_convert_type(yi, jnp.float32)
    for _ in range(3):
        y = y * (1.5 - 0.5 * x * y * y)
    return y


def _edge_body(pos_hbm, row_hbm, col_hbm,
               ux_hbm, uy_hbm, uz_hbm, part_hbm,
               pos_v, rowv, colv,
               ubx, uby, ubz, nbx, nby, nbz,
               ixr, iyr, izr, ixc, iyc, izc,
               zbuf, dtab, sem, sem2):
    c = lax.axis_index("c")
    s = lax.axis_index("s")
    wid = s * NC + c
    pltpu.sync_copy(pos_hbm, pos_v)
    zeros = jnp.zeros((LANES,), jnp.float32)
    for i in range(ZCH // LANES):
        zbuf[pl.ds(i * LANES, LANES)] = zeros
    pltpu.sync_copy(zbuf, dtab.at[pl.ds(s * ZCH, ZCH)])
    plsc.subcore_barrier()

    def block(r, carry):
        bi = wid + r * NW
        din = [
            pltpu.async_copy(row_hbm.at[pl.ds(bi * IDXROWS, IDXROWS)], rowv, sem),
            pltpu.async_copy(col_hbm.at[pl.ds(bi * IDXROWS, IDXROWS)], colv, sem),
        ]
        for d in din:
            d.wait()
        for g in range(GPB):
            j, cc = g // 8, (g % 8) * LANES
            rv = rowv[j, pl.ds(cc, LANES)]
            cv = colv[j, pl.ds(cc, LANES)]
            r3 = rv * 3
            c3 = cv * 3
            prx = plsc.load_gather(pos_v, [r3])
            pry = plsc.load_gather(pos_v, [r3 + 1])
            prz = plsc.load_gather(pos_v, [r3 + 2])
            pcx = plsc.load_gather(pos_v, [c3])
            pcy = plsc.load_gather(pos_v, [c3 + 1])
            pcz = plsc.load_gather(pos_v, [c3 + 2])
            ex = pcx - prx
            ey = pcy - pry
            ez = pcz - prz
            d2 = ex * ex + ey * ey + ez * ez
            rr = _rsqrt(jnp.maximum(d2, 1e-35))
            sn = d2 * rr
            inv = 1.0 / (sn + 1e-8)
            ux = ex * inv
            uy = ey * inv
            uz = ez * inv
            o = g * LANES
            ubx[pl.ds(o, LANES)] = ux
            uby[pl.ds(o, LANES)] = uy
            ubz[pl.ds(o, LANES)] = uz
            nbx[pl.ds(o, LANES)] = -ux
            nby[pl.ds(o, LANES)] = -uy
            nbz[pl.ds(o, LANES)] = -uz
            ixr[j, pl.ds(cc, LANES)] = rv
            iyr[j, pl.ds(cc, LANES)] = rv + NPAD
            izr[j, pl.ds(cc, LANES)] = rv + 2 * NPAD
            ixc[j, pl.ds(cc, LANES)] = cv
            iyc[j, pl.ds(cc, LANES)] = cv + NPAD
            izc[j, pl.ds(cc, LANES)] = cv + 2 * NPAD
        e0 = bi * EB
        dout = [
            pltpu.async_copy(ubx, ux_hbm.at[pl.ds(e0, EB)], sem),
            pltpu.async_copy(uby, uy_hbm.at[pl.ds(e0, EB)], sem),
            pltpu.async_copy(ubz, uz_hbm.at[pl.ds(e0, EB)], sem),
        ]
        for j in range(IDXROWS):
            sl = pl.ds(j * 128, 128)
            dsc = [
                pltpu.async_copy(ubx.at[sl], dtab.at[ixr.at[j]], sem2, add=True),
                pltpu.async_copy(uby.at[sl], dtab.at[iyr.at[j]], sem2, add=True),
                pltpu.async_copy(ubz.at[sl], dtab.at[izr.at[j]], sem2, add=True),
                pltpu.async_copy(nbx.at[sl], dtab.at[ixc.at[j]], sem2, add=True),
                pltpu.async_copy(nby.at[sl], dtab.at[iyc.at[j]], sem2, add=True),
                pltpu.async_copy(nbz.at[sl], dtab.at[izc.at[j]], sem2, add=True),
            ]
            for d in dsc:
                d.wait()
        for d in dout:
            d.wait()
        return carry

    nr = (NBLK - wid + NW - 1) // NW
    lax.fori_loop(0, nr, block, 0)
    plsc.subcore_barrier()

    @pl.when(s == 0)
    def _():
        pltpu.sync_copy(dtab, part_hbm.at[c])


_edge_call = functools.partial(
    pl.kernel,
    out_type=(
        jax.ShapeDtypeStruct((NE,), jnp.float32),
        jax.ShapeDtypeStruct((NE,), jnp.float32),
        jax.ShapeDtypeStruct((NE,), jnp.float32),
        jax.ShapeDtypeStruct((NC, DPL), jnp.float32),
    ),
    mesh=_MESH,
    compiler_params=_PARAMS,
    scratch_types=[
        pltpu.VMEM((POSPAD,), jnp.float32),
        pltpu.VMEM((IDXROWS, 128), jnp.int32),
        pltpu.VMEM((IDXROWS, 128), jnp.int32),
        pltpu.VMEM((EB,), jnp.float32),
        pltpu.VMEM((EB,), jnp.float32),
        pltpu.VMEM((EB,), jnp.float32),
        pltpu.VMEM((EB,), jnp.float32),
        pltpu.VMEM((EB,), jnp.float32),
        pltpu.VMEM((EB,), jnp.float32),
        pltpu.VMEM((IDXROWS, 128), jnp.int32),
        pltpu.VMEM((IDXROWS, 128), jnp.int32),
        pltpu.VMEM((IDXROWS, 128), jnp.int32),
        pltpu.VMEM((IDXROWS, 128), jnp.int32),
        pltpu.VMEM((IDXROWS, 128), jnp.int32),
        pltpu.VMEM((IDXROWS, 128), jnp.int32),
        pltpu.VMEM((ZCH,), jnp.float32),
        pltpu.VMEM_SHARED((DPL,), jnp.float32),
        pltpu.SemaphoreType.DMA,
        pltpu.SemaphoreType.DMA,
    ],
)(_edge_body)


NBC = 1024


def _cd_body(part_hbm, row_hbm, col_hbm, ux_hbm, uy_hbm, uz_hbm,
             dih_hbm, dunits_hbm, ang_hbm,
             p0x, p0y, p0z, p1x, p1y, p1z, db, ab, dt,
             rowv, colv, bx, by, bz, dh, dtab, sem):
    c = lax.axis_index("c")
    s = lax.axis_index("s")
    wid = s * NC + c
    base = s * NBC
    din = [
        pltpu.async_copy(part_hbm.at[0, pl.ds(base, NBC)], p0x, sem),
        pltpu.async_copy(part_hbm.at[0, pl.ds(NPAD + base, NBC)], p0y, sem),
        pltpu.async_copy(part_hbm.at[0, pl.ds(2 * NPAD + base, NBC)], p0z, sem),
        pltpu.async_copy(part_hbm.at[1, pl.ds(base, NBC)], p1x, sem),
        pltpu.async_copy(part_hbm.at[1, pl.ds(NPAD + base, NBC)], p1y, sem),
        pltpu.async_copy(part_hbm.at[1, pl.ds(2 * NPAD + base, NBC)], p1z, sem),
    ]
    for d in din:
        d.wait()
    iota = lax.iota(jnp.int32, LANES)
    for g in range(NBC // LANES):
        o = g * LANES
        sl = pl.ds(o, LANES)
        x = p0x[sl] + p1x[sl]
        y = p0y[sl] + p1y[sl]
        z = p0z[sl] + p1z[sl]
        ab[sl] = x * x + y * y + z * z
        i3 = (iota + o) * 3
        plsc.store_scatter(db, [i3], x)
        plsc.store_scatter(db, [i3 + 1], y)
        plsc.store_scatter(db, [i3 + 2], z)
    pltpu.sync_copy(db, dtab.at[pl.ds(s * 3 * NBC, 3 * NBC)])

    @pl.when(jnp.logical_and(c == 0, s < POSPAD // (3 * NBC)))
    def _():
        pltpu.sync_copy(db, dunits_hbm.at[pl.ds(s * 3 * NBC, 3 * NBC)])

    @pl.when(c == 1)
    def _():
        pltpu.sync_copy(ab, ang_hbm.at[pl.ds(base, NBC)])

    plsc.subcore_barrier()
    pltpu.sync_copy(dtab.at[pl.ds(0, POSPAD)], dt)

    def block(r, carry):
        bi = wid + r * NW
        e0 = bi * EB
        descs = [
            pltpu.async_copy(row_hbm.at[pl.ds(bi * IDXROWS, IDXROWS)], rowv, sem),
            pltpu.async_copy(col_hbm.at[pl.ds(bi * IDXROWS, IDXROWS)], colv, sem),
            pltpu.async_copy(ux_hbm.at[pl.ds(e0, EB)], bx, sem),
            pltpu.async_copy(uy_hbm.at[pl.ds(e0, EB)], by, sem),
            pltpu.async_copy(uz_hbm.at[pl.ds(e0, EB)], bz, sem),
        ]
        for d in descs:
            d.wait()
        for g in range(GPB):
            j, cc = g // 8, (g % 8) * LANES
            rv = rowv[j, pl.ds(cc, LANES)]
            cv = colv[j, pl.ds(cc, LANES)]
            r3 = rv * 3
            c3 = cv * 3
            vix = plsc.load_gather(dt, [r3])
            viy = plsc.load_gather(dt, [r3 + 1])
            viz = plsc.load_gather(dt, [r3 + 2])
            vjx = plsc.load_gather(dt, [c3])
            vjy = plsc.load_gather(dt, [c3 + 1])
            vjz = plsc.load_gather(dt, [c3 + 2])
            o = g * LANES
            ux = bx[pl.ds(o, LANES)]
            uy = by[pl.ds(o, LANES)]
            uz = bz[pl.ds(o, LANES)]
            di = vix * ux + viy * uy + viz * uz
            dj = vjx * ux + vjy * uy + vjz * uz
            dd = vix * vjx + viy * vjy + viz * vjz - di * dj
            dh[pl.ds(o, LANES)] = dd
        pltpu.sync_copy(dh, dih_hbm.at[pl.ds(e0, EB)])
        return carry

    nr = (NBLK - wid + NW - 1) // NW
    lax.fori_loop(0, nr, block, 0)


_cd_call = functools.partial(
    pl.kernel,
    out_type=(
        jax.ShapeDtypeStruct((NE,), jnp.float32),
        jax.ShapeDtypeStruct((POSPAD,), jnp.float32),
        jax.ShapeDtypeStruct((NPAD,), jnp.float32),
    ),
    mesh=_MESH,
    compiler_params=_PARAMS,
    scratch_types=[
        pltpu.VMEM((NBC,), jnp.float32),
        pltpu.VMEM((NBC,), jnp.float32),
        pltpu.VMEM((NBC,), jnp.float32),
        pltpu.VMEM((NBC,), jnp.float32),
        pltpu.VMEM((NBC,), jnp.float32),
        pltpu.VMEM((NBC,), jnp.float32),
        pltpu.VMEM((3 * NBC,), jnp.float32),
        pltpu.VMEM((NBC,), jnp.float32),
        pltpu.VMEM((POSPAD,), jnp.float32),
        pltpu.VMEM((IDXROWS, 128), jnp.int32),
        pltpu.VMEM((IDXROWS, 128), jnp.int32),
        pltpu.VMEM((EB,), jnp.float32),
        pltpu.VMEM((EB,), jnp.float32),
        pltpu.VMEM((EB,), jnp.float32),
        pltpu.VMEM((EB,), jnp.float32),
        pltpu.VMEM_SHARED((DPL,), jnp.float32),
        pltpu.SemaphoreType.DMA,
    ],
)(_cd_body)


def _bc_body(x_ref, o_ref):
    o_ref[...] = jnp.broadcast_to(x_ref[...], o_ref.shape)


def _broadcast(x, rows_per_blk):
    n = x.shape[0]
    return pl.pallas_call(
        _bc_body,
        grid=(n // rows_per_blk,),
        in_specs=[pl.BlockSpec((rows_per_blk, 1), lambda i: (i, 0))],
        out_specs=pl.BlockSpec((rows_per_blk, HID), lambda i: (i, 0)),
        out_shape=jax.ShapeDtypeStruct((n, HID), jnp.float32),
    )(x)


@jax.jit
def kernel(pos, edge_index, vector_features):
    del vector_features
    pos_flat = jnp.pad(pos.reshape(-1), (0, POSPAD - 3 * NN))
    row2d = edge_index[0].reshape(NE // 128, 128)
    col2d = edge_index[1].reshape(NE // 128, 128)
    ux, uy, uz, part = _edge_call(pos_flat, row2d, col2d)
    dih, dflat, ang = _cd_call(part, row2d, col2d, ux, uy, uz)
    angular_info = _broadcast(ang[:NN].reshape(NN, 1), 2000)
    dihedral_info = _broadcast(dih.reshape(NE, 1), 6400)
    return angular_info, dihedral_info, dflat[:3 * NN].reshape(NN, 3)


# speedup vs baseline: 1.0509x; 1.0509x over previous
import functools

import jax
import jax.numpy as jnp
from jax import lax
from jax.experimental import pallas as pl
from jax.experimental.pallas import tpu as pltpu
from jax.experimental.pallas import tpu_sc as plsc

NN = 10000
NE = 320000
HID = 128
NC, NS, LANES = 2, 16, 16
NW = NC * NS
EB = 512
NBLK = NE // EB
GPB = EB // LANES
IDXROWS = EB // 128
NPAD = 16384
DPL = 3 * NPAD
ZCH = DPL // NS
POSPAD = 30720
NB = NPAD // NW

_MESH = plsc.VectorSubcoreMesh(
    core_axis_name="c", subcore_axis_name="s", num_cores=NC, num_subcores=NS)
_PARAMS = pltpu.CompilerParams(needs_layout_passes=False)


def _rsqrt(x):
    xi = lax.bitcast_convert_type(x, jnp.int32)
    yi = jnp.int32(0x5F3759DF) - lax.shift_right_logical(xi, 1)
    y = lax.bitcast_convert_type(yi, jnp.float32)
    for _ in range(3):
        y = y * (1.5 - 0.5 * x * y * y)
    return y


def _edge_body(pos_hbm, row_hbm, col_hbm,
               ux_hbm, uy_hbm, uz_hbm, part_hbm,
               pos_v, rowv, colv,
               ubx, uby, ubz, nbx, nby, nbz,
               ixr, iyr, izr, ixc, iyc, izc,
               zbuf, dtab, sem, sem2):
    c = lax.axis_index("c")
    s = lax.axis_index("s")
    wid = s * NC + c
    pltpu.sync_copy(pos_hbm, pos_v)
    zeros = jnp.zeros((LANES,), jnp.float32)
    for i in range(ZCH // LANES):
        zbuf[pl.ds(i * LANES, LANES)] = zeros
    pltpu.sync_copy(zbuf, dtab.at[pl.ds(s * ZCH, ZCH)])
    plsc.subcore_barrier()

    def block(r, carry):
        bi = wid + r * NW
        din = [
            pltpu.async_copy(row_hbm.at[pl.ds(bi * IDXROWS, IDXROWS)], rowv, sem),
            pltpu.async_copy(col_hbm.at[pl.ds(bi * IDXROWS, IDXROWS)], colv, sem),
        ]
        for d in din:
            d.wait()
        for g in range(GPB):
            j, cc = g // 8, (g % 8) * LANES
            rv = rowv[j, pl.ds(cc, LANES)]
            cv = colv[j, pl.ds(cc, LANES)]
            r3 = rv * 3
            c3 = cv * 3
            prx = plsc.load_gather(pos_v, [r3])
            pry = plsc.load_gather(pos_v, [r3 + 1])
            prz = plsc.load_gather(pos_v, [r3 + 2])
            pcx = plsc.load_gather(pos_v, [c3])
            pcy = plsc.load_gather(pos_v, [c3 + 1])
            pcz = plsc.load_gather(pos_v, [c3 + 2])
            ex = pcx - prx
            ey = pcy - pry
            ez = pcz - prz
            d2 = ex * ex + ey * ey + ez * ez
            rr = _rsqrt(jnp.maximum(d2, 1e-35))
            sn = d2 * rr
            inv = 1.0 / (sn + 1e-8)
            ux = ex * inv
            uy = ey * inv
            uz = ez * inv
            o = g * LANES
            ubx[pl.ds(o, LANES)] = ux
            uby[pl.ds(o, LANES)] = uy
            ubz[pl.ds(o, LANES)] = uz
            nbx[pl.ds(o, LANES)] = -ux
            nby[pl.ds(o, LANES)] = -uy
            nbz[pl.ds(o, LANES)] = -uz
            ixr[j, pl.ds(cc, LANES)] = rv
            iyr[j, pl.ds(cc, LANES)] = rv + NPAD
            izr[j, pl.ds(cc, LANES)] = rv + 2 * NPAD
            ixc[j, pl.ds(cc, LANES)] = cv
            iyc[j, pl.ds(cc, LANES)] = cv + NPAD
            izc[j, pl.ds(cc, LANES)] = cv + 2 * NPAD
        e0 = bi * EB
        dout = [
            pltpu.async_copy(ubx, ux_hbm.at[pl.ds(e0, EB)], sem),
            pltpu.async_copy(uby, uy_hbm.at[pl.ds(e0, EB)], sem),
            pltpu.async_copy(ubz, uz_hbm.at[pl.ds(e0, EB)], sem),
        ]
        for j in range(IDXROWS):
            sl = pl.ds(j * 128, 128)
            dsc = [
                pltpu.async_copy(ubx.at[sl], dtab.at[ixr.at[j]], sem2, add=True),
                pltpu.async_copy(uby.at[sl], dtab.at[iyr.at[j]], sem2, add=True),
                pltpu.async_copy(ubz.at[sl], dtab.at[izr.at[j]], sem2, add=True),
                pltpu.async_copy(nbx.at[sl], dtab.at[ixc.at[j]], sem2, add=True),
                pltpu.async_copy(nby.at[sl], dtab.at[iyc.at[j]], sem2, add=True),
                pltpu.async_copy(nbz.at[sl], dtab.at[izc.at[j]], sem2, add=True),
            ]
            for d in dsc:
                d.wait()
        for d in dout:
            d.wait()
        return carry

    nr = (NBLK - wid + NW - 1) // NW
    lax.fori_loop(0, nr, block, 0)
    plsc.subcore_barrier()

    @pl.when(s == 0)
    def _():
        pltpu.sync_copy(dtab, part_hbm.at[c])


_edge_call = functools.partial(
    pl.kernel,
    out_type=(
        jax.ShapeDtypeStruct((NE,), jnp.float32),
        jax.ShapeDtypeStruct((NE,), jnp.float32),
        jax.ShapeDtypeStruct((NE,), jnp.float32),
        jax.ShapeDtypeStruct((NC, DPL), jnp.float32),
    ),
    mesh=_MESH,
    compiler_params=_PARAMS,
    scratch_types=[
        pltpu.VMEM((POSPAD,), jnp.float32),
        pltpu.VMEM((IDXROWS, 128), jnp.int32),
        pltpu.VMEM((IDXROWS, 128), jnp.int32),
        pltpu.VMEM((EB,), jnp.float32),
        pltpu.VMEM((EB,), jnp.float32),
        pltpu.VMEM((EB,), jnp.float32),
        pltpu.VMEM((EB,), jnp.float32),
        pltpu.VMEM((EB,), jnp.float32),
        pltpu.VMEM((EB,), jnp.float32),
        pltpu.VMEM((IDXROWS, 128), jnp.int32),
        pltpu.VMEM((IDXROWS, 128), jnp.int32),
        pltpu.VMEM((IDXROWS, 128), jnp.int32),
        pltpu.VMEM((IDXROWS, 128), jnp.int32),
        pltpu.VMEM((IDXROWS, 128), jnp.int32),
        pltpu.VMEM((IDXROWS, 128), jnp.int32),
        pltpu.VMEM((ZCH,), jnp.float32),
        pltpu.VMEM_SHARED((DPL,), jnp.float32),
        pltpu.SemaphoreType.DMA,
        pltpu.SemaphoreType.DMA,
    ],
)(_edge_body)


def _combine_body(part_hbm, dflat_hbm, ang_hbm,
                  p0x, p0y, p0z, p1x, p1y, p1z, db, ab):
    c = lax.axis_index("c")
    s = lax.axis_index("s")
    wid = s * NC + c
    base = wid * NB
    pltpu.sync_copy(part_hbm.at[0, pl.ds(base, NB)], p0x)
    pltpu.sync_copy(part_hbm.at[0, pl.ds(NPAD + base, NB)], p0y)
    pltpu.sync_copy(part_hbm.at[0, pl.ds(2 * NPAD + base, NB)], p0z)
    pltpu.sync_copy(part_hbm.at[1, pl.ds(base, NB)], p1x)
    pltpu.sync_copy(part_hbm.at[1, pl.ds(NPAD + base, NB)], p1y)
    pltpu.sync_copy(part_hbm.at[1, pl.ds(2 * NPAD + base, NB)], p1z)
    iota = lax.iota(jnp.int32, LANES)
    for g in range(NB // LANES):
        o = g * LANES
        sl = pl.ds(o, LANES)
        x = p0x[sl] + p1x[sl]
        y = p0y[sl] + p1y[sl]
        z = p0z[sl] + p1z[sl]
        ab[sl] = x * x + y * y + z * z
        i3 = (iota + o) * 3
        plsc.store_scatter(db, [i3], x)
        plsc.store_scatter(db, [i3 + 1], y)
        plsc.store_scatter(db, [i3 + 2], z)
    pltpu.sync_copy(ab, ang_hbm.at[pl.ds(base, NB)])

    @pl.when(wid * 3 * NB < POSPAD)
    def _():
        pltpu.sync_copy(db, dflat_hbm.at[pl.ds(wid * 3 * NB, 3 * NB)])


_combine_call = functools.partial(
    pl.kernel,
    out_type=(
        jax.ShapeDtypeStruct((POSPAD,), jnp.float32),
        jax.ShapeDtypeStruct((NPAD,), jnp.float32),
    ),
    mesh=_MESH,
    compiler_params=_PARAMS,
    scratch_types=[
        pltpu.VMEM((NB,), jnp.float32),
        pltpu.VMEM((NB,), jnp.float32),
        pltpu.VMEM((NB,), jnp.float32),
        pltpu.VMEM((NB,), jnp.float32),
        pltpu.VMEM((NB,), jnp.float32),
        pltpu.VMEM((NB,), jnp.float32),
        pltpu.VMEM((3 * NB,), jnp.float32),
        pltpu.VMEM((NB,), jnp.float32),
    ],
)(_combine_body)


def _dih_body(dflat_hbm, row_hbm, col_hbm, ux_hbm, uy_hbm, uz_hbm,
              dih_hbm, dt, rowv, colv, bx, by, bz, dh, sem):
    c = lax.axis_index("c")
    s = lax.axis_index("s")
    wid = s * NC + c
    pltpu.sync_copy(dflat_hbm, dt)

    def block(r, carry):
        bi = wid + r * NW
        e0 = bi * EB
        descs = [
            pltpu.async_copy(row_hbm.at[pl.ds(bi * IDXROWS, IDXROWS)], rowv, sem),
            pltpu.async_copy(col_hbm.at[pl.ds(bi * IDXROWS, IDXROWS)], colv, sem),
            pltpu.async_copy(ux_hbm.at[pl.ds(e0, EB)], bx, sem),
            pltpu.async_copy(uy_hbm.at[pl.ds(e0, EB)], by, sem),
            pltpu.async_copy(uz_hbm.at[pl.ds(e0, EB)], bz, sem),
        ]
        for d in descs:
            d.wait()
        for g in range(GPB):
            j, cc = g // 8, (g % 8) * LANES
            rv = rowv[j, pl.ds(cc, LANES)]
            cv = colv[j, pl.ds(cc, LANES)]
            r3 = rv * 3
            c3 = cv * 3
            vix = plsc.load_gather(dt, [r3])
            viy = plsc.load_gather(dt, [r3 + 1])
            viz = plsc.load_gather(dt, [r3 + 2])
            vjx = plsc.load_gather(dt, [c3])
            vjy = plsc.load_gather(dt, [c3 + 1])
            vjz = plsc.load_gather(dt, [c3 + 2])
            o = g * LANES
            ux = bx[pl.ds(o, LANES)]
            uy = by[pl.ds(o, LANES)]
            uz = bz[pl.ds(o, LANES)]
            di = vix * ux + viy * uy + viz * uz
            dj = vjx * ux + vjy * uy + vjz * uz
            dd = vix * vjx + viy * vjy + viz * vjz - di * dj
            dh[pl.ds(o, LANES)] = dd
        pltpu.sync_copy(dh, dih_hbm.at[pl.ds(e0, EB)])
        return carry

    nr = (NBLK - wid + NW - 1) // NW
    lax.fori_loop(0, nr, block, 0)


_dih_call = functools.partial(
    pl.kernel,
    out_type=jax.ShapeDtypeStruct((NE,), jnp.float32),
    mesh=_MESH,
    compiler_params=_PARAMS,
    scratch_types=[
        pltpu.VMEM((POSPAD,), jnp.float32),
        pltpu.VMEM((IDXROWS, 128), jnp.int32),
        pltpu.VMEM((IDXROWS, 128), jnp.int32),
        pltpu.VMEM((EB,), jnp.float32),
        pltpu.VMEM((EB,), jnp.float32),
        pltpu.VMEM((EB,), jnp.float32),
        pltpu.VMEM((EB,), jnp.float32),
        pltpu.SemaphoreType.DMA,
    ],
)(_dih_body)


def _bc_body(x_ref, o_ref):
    o_ref[...] = jnp.broadcast_to(x_ref[...], o_ref.shape)


def _broadcast(x, rows_per_blk):
    n = x.shape[0]
    return pl.pallas_call(
        _bc_body,
        grid=(n // rows_per_blk,),
        in_specs=[pl.BlockSpec((rows_per_blk, 1), lambda i: (i, 0))],
        out_specs=pl.BlockSpec((rows_per_blk, HID), lambda i: (i, 0)),
        out_shape=jax.ShapeDtypeStruct((n, HID), jnp.float32),
    )(x)


@jax.jit
def kernel(pos, edge_index, vector_features):
    del vector_features
    pos_flat = jnp.pad(pos.reshape(-1), (0, POSPAD - 3 * NN))
    row2d = edge_index[0].reshape(NE // 128, 128)
    col2d = edge_index[1].reshape(NE // 128, 128)
    ux, uy, uz, part = _edge_call(pos_flat, row2d, col2d)
    dflat, ang = _combine_call(part)
    dih = _dih_call(dflat, row2d, col2d, ux, uy, uz)
    angular_info = _broadcast(ang[:NN].reshape(NN, 1), 2000)
    dihedral_info = _broadcast(dih.reshape(NE, 1), 12800)
    return angular_info, dihedral_info, dflat[:3 * NN].reshape(NN, 3)


# speedup vs baseline: 1.0562x; 1.0051x over previous
import functools

import jax
import jax.numpy as jnp
from jax import lax
from jax.experimental import pallas as pl
from jax.experimental.pallas import tpu as pltpu
from jax.experimental.pallas import tpu_sc as plsc

NN = 10000
NE = 320000
HID = 128
NC, NS, LANES = 2, 16, 16
NW = NC * NS
EB = 512
NBLK = NE // EB
GPB = EB // LANES
IDXROWS = EB // 128
NPAD = 16384
DPL = 3 * NPAD
ZCH = DPL // NS
POSPAD = 30720
NB = NPAD // NW

_MESH = plsc.VectorSubcoreMesh(
    core_axis_name="c", subcore_axis_name="s", num_cores=NC, num_subcores=NS)
_PARAMS = pltpu.CompilerParams(needs_layout_passes=False)


def _rsqrt(x):
    xi = lax.bitcast_convert_type(x, jnp.int32)
    yi = jnp.int32(0x5F3759DF) - lax.shift_right_logical(xi, 1)
    y = lax.bitcast_convert_type(yi, jnp.float32)
    for _ in range(3):
        y = y * (1.5 - 0.5 * x * y * y)
    return y


def _edge_body(pos_hbm, row_hbm, col_hbm,
               ux_hbm, uy_hbm, uz_hbm, part_hbm,
               pos_v, rowv, colv,
               ubx, uby, ubz, nbx, nby, nbz,
               ixr, iyr, izr, ixc, iyc, izc,
               zbuf, dtab, sem, sem2):
    c = lax.axis_index("c")
    s = lax.axis_index("s")
    wid = s * NC + c
    pltpu.sync_copy(pos_hbm, pos_v)
    zeros = jnp.zeros((LANES,), jnp.float32)
    for i in range(ZCH // LANES):
        zbuf[pl.ds(i * LANES, LANES)] = zeros
    pltpu.sync_copy(zbuf, dtab.at[pl.ds(s * ZCH, ZCH)])
    plsc.subcore_barrier()

    def block(r, carry):
        bi = wid + r * NW
        din = [
            pltpu.async_copy(row_hbm.at[pl.ds(bi * IDXROWS, IDXROWS)], rowv, sem),
            pltpu.async_copy(col_hbm.at[pl.ds(bi * IDXROWS, IDXROWS)], colv, sem),
        ]
        for d in din:
            d.wait()
        for g in range(GPB):
            j, cc = g // 8, (g % 8) * LANES
            rv = rowv[j, pl.ds(cc, LANES)]
            cv = colv[j, pl.ds(cc, LANES)]
            r3 = rv * 3
            c3 = cv * 3
            prx = plsc.load_gather(pos_v, [r3])
            pry = plsc.load_gather(pos_v, [r3 + 1])
            prz = plsc.load_gather(pos_v, [r3 + 2])
            pcx = plsc.load_gather(pos_v, [c3])
            pcy = plsc.load_gather(pos_v, [c3 + 1])
            pcz = plsc.load_gather(pos_v, [c3 + 2])
            ex = pcx - prx
            ey = pcy - pry
            ez = pcz - prz
            d2 = ex * ex + ey * ey + ez * ez
            rr = _rsqrt(jnp.maximum(d2, 1e-35))
            sn = d2 * rr
            inv = 1.0 / (sn + 1e-8)
            ux = ex * inv
            uy = ey * inv
            uz = ez * inv
            o = g * LANES
            ubx[pl.ds(o, LANES)] = ux
            uby[pl.ds(o, LANES)] = uy
            ubz[pl.ds(o, LANES)] = uz
            nbx[pl.ds(o, LANES)] = -ux
            nby[pl.ds(o, LANES)] = -uy
            nbz[pl.ds(o, LANES)] = -uz
            ixr[j, pl.ds(cc, LANES)] = rv
            iyr[j, pl.ds(cc, LANES)] = rv + NPAD
            izr[j, pl.ds(cc, LANES)] = rv + 2 * NPAD
            ixc[j, pl.ds(cc, LANES)] = cv
            iyc[j, pl.ds(cc, LANES)] = cv + NPAD
            izc[j, pl.ds(cc, LANES)] = cv + 2 * NPAD
        e0 = bi * EB
        dout = [
            pltpu.async_copy(ubx, ux_hbm.at[pl.ds(e0, EB)], sem),
            pltpu.async_copy(uby, uy_hbm.at[pl.ds(e0, EB)], sem),
            pltpu.async_copy(ubz, uz_hbm.at[pl.ds(e0, EB)], sem),
        ]
        dsc = []
        for j in range(IDXROWS):
            sl = pl.ds(j * 128, 128)
            dsc.append(pltpu.async_copy(ubx.at[sl], dtab.at[ixr.at[j]], sem2, add=True))
            dsc.append(pltpu.async_copy(uby.at[sl], dtab.at[iyr.at[j]], sem2, add=True))
            dsc.append(pltpu.async_copy(ubz.at[sl], dtab.at[izr.at[j]], sem2, add=True))
            dsc.append(pltpu.async_copy(nbx.at[sl], dtab.at[ixc.at[j]], sem2, add=True))
            dsc.append(pltpu.async_copy(nby.at[sl], dtab.at[iyc.at[j]], sem2, add=True))
            dsc.append(pltpu.async_copy(nbz.at[sl], dtab.at[izc.at[j]], sem2, add=True))
        for d in dsc:
            d.wait()
        for d in dout:
            d.wait()
        return carry

    nr = (NBLK - wid + NW - 1) // NW
    lax.fori_loop(0, nr, block, 0)
    plsc.subcore_barrier()

    @pl.when(s == 0)
    def _():
        pltpu.sync_copy(dtab, part_hbm.at[c])


_edge_call = functools.partial(
    pl.kernel,
    out_type=(
        jax.ShapeDtypeStruct((NE,), jnp.float32),
        jax.ShapeDtypeStruct((NE,), jnp.float32),
        jax.ShapeDtypeStruct((NE,), jnp.float32),
        jax.ShapeDtypeStruct((NC, DPL), jnp.float32),
    ),
    mesh=_MESH,
    compiler_params=_PARAMS,
    scratch_types=[
        pltpu.VMEM((POSPAD,), jnp.float32),
        pltpu.VMEM((IDXROWS, 128), jnp.int32),
        pltpu.VMEM((IDXROWS, 128), jnp.int32),
        pltpu.VMEM((EB,), jnp.float32),
        pltpu.VMEM((EB,), jnp.float32),
        pltpu.VMEM((EB,), jnp.float32),
        pltpu.VMEM((EB,), jnp.float32),
        pltpu.VMEM((EB,), jnp.float32),
        pltpu.VMEM((EB,), jnp.float32),
        pltpu.VMEM((IDXROWS, 128), jnp.int32),
        pltpu.VMEM((IDXROWS, 128), jnp.int32),
        pltpu.VMEM((IDXROWS, 128), jnp.int32),
        pltpu.VMEM((IDXROWS, 128), jnp.int32),
        pltpu.VMEM((IDXROWS, 128), jnp.int32),
        pltpu.VMEM((IDXROWS, 128), jnp.int32),
        pltpu.VMEM((ZCH,), jnp.float32),
        pltpu.VMEM_SHARED((DPL,), jnp.float32),
        pltpu.SemaphoreType.DMA,
        pltpu.SemaphoreType.DMA,
    ],
)(_edge_body)


def _combine_body(part_hbm, dflat_hbm, ang_hbm,
                  p0x, p0y, p0z, p1x, p1y, p1z, db, ab):
    c = lax.axis_index("c")
    s = lax.axis_index("s")
    wid = s * NC + c
    base = wid * NB
    pltpu.sync_copy(part_hbm.at[0, pl.ds(base, NB)], p0x)
    pltpu.sync_copy(part_hbm.at[0, pl.ds(NPAD + base, NB)], p0y)
    pltpu.sync_copy(part_hbm.at[0, pl.ds(2 * NPAD + base, NB)], p0z)
    pltpu.sync_copy(part_hbm.at[1, pl.ds(base, NB)], p1x)
    pltpu.sync_copy(part_hbm.at[1, pl.ds(NPAD + base, NB)], p1y)
    pltpu.sync_copy(part_hbm.at[1, pl.ds(2 * NPAD + base, NB)], p1z)
    iota = lax.iota(jnp.int32, LANES)
    for g in range(NB // LANES):
        o = g * LANES
        sl = pl.ds(o, LANES)
        x = p0x[sl] + p1x[sl]
        y = p0y[sl] + p1y[sl]
        z = p0z[sl] + p1z[sl]
        ab[sl] = x * x + y * y + z * z
        i3 = (iota + o) * 3
        plsc.store_scatter(db, [i3], x)
        plsc.store_scatter(db, [i3 + 1], y)
        plsc.store_scatter(db, [i3 + 2], z)
    pltpu.sync_copy(ab, ang_hbm.at[pl.ds(base, NB)])

    @pl.when(wid * 3 * NB < POSPAD)
    def _():
        pltpu.sync_copy(db, dflat_hbm.at[pl.ds(wid * 3 * NB, 3 * NB)])


_combine_call = functools.partial(
    pl.kernel,
    out_type=(
        jax.ShapeDtypeStruct((POSPAD,), jnp.float32),
        jax.ShapeDtypeStruct((NPAD,), jnp.float32),
    ),
    mesh=_MESH,
    compiler_params=_PARAMS,
    scratch_types=[
        pltpu.VMEM((NB,), jnp.float32),
        pltpu.VMEM((NB,), jnp.float32),
        pltpu.VMEM((NB,), jnp.float32),
        pltpu.VMEM((NB,), jnp.float32),
        pltpu.VMEM((NB,), jnp.float32),
        pltpu.VMEM((NB,), jnp.float32),
        pltpu.VMEM((3 * NB,), jnp.float32),
        pltpu.VMEM((NB,), jnp.float32),
    ],
)(_combine_body)


def _dih_body(dflat_hbm, row_hbm, col_hbm, ux_hbm, uy_hbm, uz_hbm,
              dih_hbm, dt, rowv, colv, bx, by, bz, dh, sem):
    c = lax.axis_index("c")
    s = lax.axis_index("s")
    wid = s * NC + c
    pltpu.sync_copy(dflat_hbm, dt)

    def block(r, carry):
        bi = wid + r * NW
        e0 = bi * EB
        descs = [
            pltpu.async_copy(row_hbm.at[pl.ds(bi * IDXROWS, IDXROWS)], rowv, sem),
            pltpu.async_copy(col_hbm.at[pl.ds(bi * IDXROWS, IDXROWS)], colv, sem),
            pltpu.async_copy(ux_hbm.at[pl.ds(e0, EB)], bx, sem),
            pltpu.async_copy(uy_hbm.at[pl.ds(e0, EB)], by, sem),
            pltpu.async_copy(uz_hbm.at[pl.ds(e0, EB)], bz, sem),
        ]
        for d in descs:
            d.wait()
        for g in range(GPB):
            j, cc = g // 8, (g % 8) * LANES
            rv = rowv[j, pl.ds(cc, LANES)]
            cv = colv[j, pl.ds(cc, LANES)]
            r3 = rv * 3
            c3 = cv * 3
            vix = plsc.load_gather(dt, [r3])
            viy = plsc.load_gather(dt, [r3 + 1])
            viz = plsc.load_gather(dt, [r3 + 2])
            vjx = plsc.load_gather(dt, [c3])
            vjy = plsc.load_gather(dt, [c3 + 1])
            vjz = plsc.load_gather(dt, [c3 + 2])
            o = g * LANES
            ux = bx[pl.ds(o, LANES)]
            uy = by[pl.ds(o, LANES)]
            uz = bz[pl.ds(o, LANES)]
            di = vix * ux + viy * uy + viz * uz
            dj = vjx * ux + vjy * uy + vjz * uz
            dd = vix * vjx + viy * vjy + viz * vjz - di * dj
            dh[pl.ds(o, LANES)] = dd
        pltpu.sync_copy(dh, dih_hbm.at[pl.ds(e0, EB)])
        return carry

    nr = (NBLK - wid + NW - 1) // NW
    lax.fori_loop(0, nr, block, 0)


_dih_call = functools.partial(
    pl.kernel,
    out_type=jax.ShapeDtypeStruct((NE,), jnp.float32),
    mesh=_MESH,
    compiler_params=_PARAMS,
    scratch_types=[
        pltpu.VMEM((POSPAD,), jnp.float32),
        pltpu.VMEM((IDXROWS, 128), jnp.int32),
        pltpu.VMEM((IDXROWS, 128), jnp.int32),
        pltpu.VMEM((EB,), jnp.float32),
        pltpu.VMEM((EB,), jnp.float32),
        pltpu.VMEM((EB,), jnp.float32),
        pltpu.VMEM((EB,), jnp.float32),
        pltpu.SemaphoreType.DMA,
    ],
)(_dih_body)


def _bc_body(x_ref, o_ref):
    o_ref[...] = jnp.broadcast_to(x_ref[...], o_ref.shape)


def _broadcast(x, rows_per_blk):
    n = x.shape[0]
    return pl.pallas_call(
        _bc_body,
        grid=(n // rows_per_blk,),
        in_specs=[pl.BlockSpec((rows_per_blk, 1), lambda i: (i, 0))],
        out_specs=pl.BlockSpec((rows_per_blk, HID), lambda i: (i, 0)),
        out_shape=jax.ShapeDtypeStruct((n, HID), jnp.float32),
    )(x)


@jax.jit
def kernel(pos, edge_index, vector_features):
    del vector_features
    pos_flat = jnp.pad(pos.reshape(-1), (0, POSPAD - 3 * NN))
    row2d = edge_index[0].reshape(NE // 128, 128)
    col2d = edge_index[1].reshape(NE // 128, 128)
    ux, uy, uz, part = _edge_call(pos_flat, row2d, col2d)
    dflat, ang = _combine_call(part)
    dih = _dih_call(dflat, row2d, col2d, ux, uy, uz)
    angular_info = _broadcast(ang[:NN].reshape(NN, 1), 2000)
    dihedral_info = _broadcast(dih.reshape(NE, 1), 12800)
    return angular_info, dihedral_info, dflat[:3 * NN].reshape(NN, 3)
